# baseline (speedup 1.0000x reference)
import jax
import jax.numpy as jnp
from jax import lax
from jax.experimental import pallas as pl
from jax.experimental.pallas import tpu as pltpu
from jax.experimental.pallas import tpu_sc as plsc

D = 64
S = 200
NW = 32
SUB = 100
SEQ_PER_CHUNK = 2
CHUNK = SEQ_PER_CHUNK * S
SUBS_PER_CHUNK = CHUNK // SUB


def _body(xf_hbm, table_hbm, pe_hbm, out_hbm,
          idx_v, pe_v, rows0, rows1, gsem0, gsem1):
    nsub = xf_hbm.shape[1]
    rows_per_worker = nsub * SUB
    nchunk = rows_per_worker // CHUNK

    wid = lax.axis_index("s") * 2 + lax.axis_index("c")
    base = wid * rows_per_worker

    pltpu.sync_copy(xf_hbm.at[wid], idx_v)
    pltpu.sync_copy(pe_hbm, pe_v)

    def gather_chunk(c, buf, sem):
        for k in range(SUBS_PER_CHUNK):
            pltpu.async_copy(
                table_hbm.at[idx_v.at[c * SUBS_PER_CHUNK + k]],
                buf.at[pl.ds(k * SUB, SUB)],
                sem,
            )

    def drain_chunk(buf, sem):
        for k in range(SUBS_PER_CHUNK):
            pltpu.make_async_copy(
                table_hbm.at[idx_v.at[k]],
                buf.at[pl.ds(k * SUB, SUB)],
                sem,
            ).wait()

    def add_pe(buf):
        def jbody(j, carry):
            for c2 in range(SEQ_PER_CHUNK):
                for k in range(D // 16):
                    pv = pe_v[j, pl.ds(k * 16, 16)]
                    plsc.addupdate(buf.at[c2 * S + j, pl.ds(k * 16, 16)], pv)
            return carry
        lax.fori_loop(0, S, jbody, 0)

    gather_chunk(0, rows0, gsem0)

    def pair_body(t, carry):
        c0 = 2 * t
        gather_chunk(c0 + 1, rows1, gsem1)
        drain_chunk(rows0, gsem0)
        add_pe(rows0)
        pltpu.sync_copy(rows0, out_hbm.at[pl.ds(base + c0 * CHUNK, CHUNK)])

        @pl.when(t < nchunk // 2 - 1)
        def _():
            gather_chunk(c0 + 2, rows0, gsem0)

        drain_chunk(rows1, gsem1)
        add_pe(rows1)
        pltpu.sync_copy(rows1, out_hbm.at[pl.ds(base + (c0 + 1) * CHUNK, CHUNK)])
        return carry

    lax.fori_loop(0, nchunk // 2, pair_body, 0)


def kernel(x, table, pe):
    b, s = x.shape
    rows = b * s
    nsub = rows // (NW * SUB)
    xf = x.reshape(NW, nsub, SUB)
    pe_s = pe[:s]

    mesh = plsc.VectorSubcoreMesh(core_axis_name="c", subcore_axis_name="s")
    out = pl.kernel(
        _body,
        out_type=jax.ShapeDtypeStruct((rows, D), jnp.float32),
        mesh=mesh,
        compiler_params=pltpu.CompilerParams(use_tc_tiling_on_sc=False),
        scratch_types=[
            pltpu.VMEM((nsub, SUB), jnp.int32),
            pltpu.VMEM((S, D), jnp.float32),
            pltpu.VMEM((CHUNK, D), jnp.float32),
            pltpu.VMEM((CHUNK, D), jnp.float32),
            pltpu.SemaphoreType.DMA,
            pltpu.SemaphoreType.DMA,
        ],
    )(xf, table, pe_s)
    return out.reshape(b, s, D)
